# baseline (speedup 1.0000x reference)
import functools

import jax
import jax.numpy as jnp
from jax import lax
from jax.experimental import pallas as pl
from jax.experimental.pallas import tpu as pltpu
from jax.experimental.pallas import tpu_sc as plsc

_N = 10000
_E = 160000
_D_HALF = 128
_K = 128
_TILES = 16
_CHUNKS = 80
_EPAD = _TILES * _CHUNKS * _K
_NIDX = 4
_BLK = 200
_NBLK = _N // _BLK
_MAXB = 4


def _spmm_sc(combo, vals, xa, xb):
    mesh = plsc.VectorSubcoreMesh(core_axis_name="c", subcore_axis_name="s")

    @functools.partial(
        pl.kernel,
        mesh=mesh,
        compiler_params=pltpu.CompilerParams(
            needs_layout_passes=False, use_tc_tiling_on_sc=False),
        out_type=(
            jax.ShapeDtypeStruct((_N, _D_HALF), jnp.float32),
            jax.ShapeDtypeStruct((_N, _D_HALF), jnp.float32),
        ),
        scratch_types=[
            pltpu.VMEM((_NIDX, 2, _K), jnp.int32),
            pltpu.VMEM((_NIDX, _K), jnp.float32),
            pltpu.VMEM((_K, _D_HALF // 2), jnp.int32),
            pltpu.VMEM((_K, _D_HALF // 2), jnp.int32),
            pltpu.VMEM((_K, _D_HALF), jnp.float32),
            pltpu.VMEM((_K, _D_HALF), jnp.float32),
            pltpu.VMEM_SHARED((_N, _D_HALF), jnp.float32),
            pltpu.SemaphoreType.DMA((2,)),
            pltpu.SemaphoreType.DMA((2,)),
        ],
    )
    def k(combo_h, vals_h, xa_h, xb_h, oa_h, ob_h,
          idx_t, val_t, g0_v, g1_v, o0_v, o1_v, acc, gsem, ssem):
        c = lax.axis_index("c")
        s = lax.axis_index("s")
        gbuf = (g0_v, g1_v)
        obuf = (o0_v, o1_v)

        zeros16 = jnp.zeros((16,), jnp.float32)

        def zrow(i, _):
            for j in range(_D_HALF // 16):
                obuf[0][i, pl.ds(j * 16, 16)] = zeros16
            return _

        lax.fori_loop(0, _K, zrow, None)

        def zinit(i, _):
            blk = s + _TILES * i

            @pl.when(blk < _NBLK)
            def _z():
                base = blk * _BLK
                pltpu.sync_copy(obuf[0], acc.at[pl.ds(base, _K)])
                pltpu.sync_copy(obuf[0].at[pl.ds(0, _BLK - _K)],
                                acc.at[pl.ds(base + _K, _BLK - _K)])
            return _

        lax.fori_loop(0, _MAXB, zinit, None)
        plsc.subcore_barrier()

        def load_idx(ci, q):
            pltpu.sync_copy(combo_h.at[s, ci], idx_t.at[q])
            pltpu.sync_copy(vals_h.at[s, ci], val_t.at[q])

        def gather(b, q):
            @pl.when(c == 0)
            def _g0():
                pltpu.async_copy(xa_h.at[idx_t.at[q, 0]], gbuf[b], gsem.at[b])

            @pl.when(c == 1)
            def _g1():
                pltpu.async_copy(xb_h.at[idx_t.at[q, 0]], gbuf[b], gsem.at[b])

        def gather_wait(b, q):
            @pl.when(c == 0)
            def _w0():
                pltpu.make_async_copy(
                    xa_h.at[idx_t.at[q, 0]], gbuf[b], gsem.at[b]).wait()

            @pl.when(c == 1)
            def _w1():
                pltpu.make_async_copy(
                    xb_h.at[idx_t.at[q, 0]], gbuf[b], gsem.at[b]).wait()

        def scatter_desc(b, q):
            return pltpu.make_async_copy(
                obuf[b], acc.at[idx_t.at[q, 1]], ssem.at[b])

        def scale(b, q):
            def edge_group(g, _):
                vv = val_t[q, pl.ds(g * 16, 16)]
                for l in range(16):
                    v = vv[l]
                    e = g * 16 + l
                    for j in range(_D_HALF // 32):
                        w = gbuf[b][e, pl.ds(j * 16, 16)]
                        vb = plsc.bitcast(w, jnp.bfloat16)
                        lo, hi = plsc.unpack(
                            vb, format=plsc.PackFormat.INTERLEAVED)
                        obuf[b][e, pl.ds(j * 32, 16)] = lo * v
                        obuf[b][e, pl.ds(j * 32 + 16, 16)] = hi * v
                return _

            lax.fori_loop(0, _K // 16, edge_group, None)

        load_idx(0, 0)
        gather(0, 0)
        load_idx(1, 1)
        gather(1, 1)

        def quad(i, _):
            for kk in range(4):
                ci = 4 * i + kk
                b = kk % 2
                q = kk
                q2 = (kk + 2) % 4

                gather_wait(b, q)

                @pl.when(ci >= 2)
                def _ws():
                    scatter_desc(b, q2).wait()

                @pl.when(ci + 2 < _CHUNKS)
                def _li():
                    load_idx(ci + 2, q2)

                scale(b, q)

                @pl.when(ci + 2 < _CHUNKS)
                def _gi():
                    gather(b, q2)

                scatter_desc(b, q).start(add=True)
            return _

        lax.fori_loop(0, _CHUNKS // 4, quad, None)
        scatter_desc(0, 2).wait()
        scatter_desc(1, 3).wait()
        plsc.subcore_barrier()

        def epi(i, _):
            blk = s + _TILES * i

            @pl.when(blk < _NBLK)
            def _e():
                sl = pl.ds(blk * _BLK, _BLK)

                @pl.when(c == 0)
                def _w0():
                    pltpu.sync_copy(acc.at[sl], oa_h.at[sl])

                @pl.when(c == 1)
                def _w1():
                    pltpu.sync_copy(acc.at[sl], ob_h.at[sl])
            return _

        lax.fori_loop(0, _MAXB, epi, None)

    return k(combo, vals, xa, xb)


def _dense_tc(sa, sb, W1a, W1b, b1r, W2a, W2b):
    bm = 1000

    def body(sa_r, sb_r, w1a_r, w1b_r, b1_r, w2a_r, w2b_r, ta_r, tb_r):
        h = jnp.dot(sa_r[...], w1a_r[...], preferred_element_type=jnp.float32)
        h = h + jnp.dot(sb_r[...], w1b_r[...], preferred_element_type=jnp.float32)
        h = jnp.maximum(h + b1_r[...], 0.0)
        ta_r[...] = jnp.dot(h, w2a_r[...], preferred_element_type=jnp.float32)
        tb_r[...] = jnp.dot(h, w2b_r[...], preferred_element_type=jnp.float32)

    hid = W1a.shape[1]
    return pl.pallas_call(
        body,
        grid=(_N // bm,),
        in_specs=[
            pl.BlockSpec((bm, _D_HALF), lambda i: (i, 0)),
            pl.BlockSpec((bm, _D_HALF), lambda i: (i, 0)),
            pl.BlockSpec((_D_HALF, hid), lambda i: (0, 0)),
            pl.BlockSpec((_D_HALF, hid), lambda i: (0, 0)),
            pl.BlockSpec((1, hid), lambda i: (0, 0)),
            pl.BlockSpec((hid, _D_HALF), lambda i: (0, 0)),
            pl.BlockSpec((hid, _D_HALF), lambda i: (0, 0)),
        ],
        out_specs=[
            pl.BlockSpec((bm, _D_HALF), lambda i: (i, 0)),
            pl.BlockSpec((bm, _D_HALF), lambda i: (i, 0)),
        ],
        out_shape=[
            jax.ShapeDtypeStruct((_N, _D_HALF), jnp.float32),
            jax.ShapeDtypeStruct((_N, _D_HALF), jnp.float32),
        ],
    )(sa, sb, W1a, W1b, b1r, W2a, W2b)


def _bf16_interleave(m):
    n = m.shape[0]
    s = (m.reshape(n, _D_HALF // 32, 2, 16).transpose(0, 1, 3, 2)
         .reshape(n, _D_HALF // 2, 2).astype(jnp.bfloat16))
    return jax.lax.bitcast_convert_type(s, jnp.int32)


def kernel(x, adj_vals, edge_index, W1, b1, W2, b2):
    src = edge_index[0].astype(jnp.int32)
    dst = edge_index[1].astype(jnp.int32)
    pad = _EPAD - _E
    shape3 = (_TILES, _CHUNKS, _K)
    src = jnp.concatenate([src, jnp.zeros((pad,), jnp.int32)]).reshape(shape3)
    dst = jnp.concatenate([dst, jnp.zeros((pad,), jnp.int32)]).reshape(shape3)
    vals = jnp.concatenate(
        [adj_vals, jnp.zeros((pad,), jnp.float32)]).reshape(shape3)
    combo = jnp.stack([src, dst], axis=2)

    xa = _bf16_interleave(x[:, :_D_HALF])
    xb = _bf16_interleave(x[:, _D_HALF:])
    sa, sb = _spmm_sc(combo, vals, xa, xb)

    ta, tb = _dense_tc(sa, sb, W1[:_D_HALF], W1[_D_HALF:],
                       b1.reshape(1, -1), W2[:, :_D_HALF], W2[:, _D_HALF:])

    oa, ob = _spmm_sc(combo, vals,
                      _bf16_interleave(ta), _bf16_interleave(tb))
    return jnp.concatenate([oa, ob], axis=1) + b2
